# baseline (speedup 1.0000x reference)
import functools

import jax
import jax.numpy as jnp
from jax import lax
from jax.experimental import pallas as pl
from jax.experimental.pallas import tpu as pltpu
from jax.experimental.pallas import tpu_sc as plsc

N = 10000
H = 128
NPAD = 10112
E = 320000
NC, NS = 2, 16
EPS = E // NS
CH = 125
NCHUNK = EPS // CH
HC = NCHUNK // 2
RPS = NPAD // NS
CPAD = 10240
CW = CPAD // NS

_MESH = plsc.VectorSubcoreMesh(core_axis_name="c", subcore_axis_name="s")


def _run_edge_type(with_counts, sid, x_hbm, src_hbm, dst_hbm, sums_out,
                   cnt_out, src_v, dst_v, rows_v, ones_v, cnt_sh, acc_sh,
                   sem):
    zero16 = jnp.zeros((16,), jnp.float32)
    for r in range(CH):
        for k in range(H // 16):
            rows_v[r, pl.ds(k * 16, 16)] = zero16
    ZC = 120
    for b in range(RPS // ZC):
        pltpu.sync_copy(rows_v.at[pl.ds(0, ZC)],
                        acc_sh.at[pl.ds(sid * RPS + b * ZC, ZC)])
    tail = RPS % ZC
    if tail:
        pltpu.sync_copy(rows_v.at[pl.ds(0, tail)],
                        acc_sh.at[pl.ds(sid * RPS + (RPS // ZC) * ZC, tail)])

    if with_counts:
        one16 = jnp.ones((16,), jnp.float32)
        for k in range(8):
            ones_v[pl.ds(k * 16, 16)] = one16
        for b in range(CW // H):
            pltpu.sync_copy(rows_v.at[2 * b],
                            cnt_sh.at[pl.ds(sid * CW + b * H, H)])

    plsc.subcore_barrier()

    for half in range(2):
        pltpu.sync_copy(src_hbm.at[sid, pl.ds(half * HC, HC)], src_v)
        pltpu.sync_copy(dst_hbm.at[sid, pl.ds(half * HC, HC)], dst_v)

        def step(j, carry):
            pltpu.async_copy(x_hbm.at[src_v.at[j]], rows_v, sem).wait()
            pltpu.sync_copy(rows_v, acc_sh.at[dst_v.at[j]], add=True)
            if with_counts:
                pltpu.sync_copy(ones_v.at[pl.ds(0, CH)],
                                cnt_sh.at[dst_v.at[j]], add=True)
            return carry
        lax.fori_loop(0, HC, step, 0)

    plsc.subcore_barrier()

    pltpu.sync_copy(acc_sh.at[pl.ds(sid * RPS, RPS)],
                    sums_out.at[pl.ds(sid * RPS, RPS)])
    if with_counts:
        pltpu.sync_copy(cnt_sh.at[pl.ds(sid * CW, CW)],
                        cnt_out.at[pl.ds(sid * CW, CW)])


def _agg_body(with_counts, x_u_hbm, x_i_hbm, src_ui, dst_ui, src_iu, dst_iu,
              sums_i_out, sums_u_out, *rest):
    if with_counts:
        (cnt_i_out, cnt_u_out, src_v, dst_v, rows_v, ones_v, sem,
         cnt_sh, acc_sh) = rest
    else:
        cnt_i_out = cnt_u_out = cnt_sh = None
        (src_v, dst_v, rows_v, ones_v, sem, acc_sh) = rest
    cid = lax.axis_index("c")
    sid = lax.axis_index("s")

    @pl.when(cid == 0)
    def _():
        _run_edge_type(with_counts, sid, x_u_hbm, src_ui, dst_ui,
                       sums_i_out, cnt_i_out, src_v, dst_v, rows_v, ones_v,
                       cnt_sh, acc_sh, sem)

    @pl.when(cid == 1)
    def _():
        _run_edge_type(with_counts, sid, x_i_hbm, src_iu, dst_iu,
                       sums_u_out, cnt_u_out, src_v, dst_v, rows_v, ones_v,
                       cnt_sh, acc_sh, sem)


def _make_agg(with_counts):
    out_type = [jax.ShapeDtypeStruct((NPAD, H), jnp.float32),
                jax.ShapeDtypeStruct((NPAD, H), jnp.float32)]
    if with_counts:
        out_type += [jax.ShapeDtypeStruct((CPAD,), jnp.float32),
                     jax.ShapeDtypeStruct((CPAD,), jnp.float32)]
    scratch = [
        pltpu.VMEM((HC, CH), jnp.int32),
        pltpu.VMEM((HC, CH), jnp.int32),
        pltpu.VMEM((CH, H), jnp.float32),
        pltpu.VMEM((128,), jnp.float32),
        pltpu.SemaphoreType.DMA,
    ]
    if with_counts:
        scratch.append(pltpu.VMEM_SHARED((CPAD,), jnp.float32))
    scratch.append(pltpu.VMEM_SHARED((NPAD, H), jnp.float32))
    return pl.kernel(
        functools.partial(_agg_body, with_counts),
        out_type=tuple(out_type),
        mesh=_MESH,
        scratch_types=tuple(scratch),
    )


_agg_with_counts = _make_agg(True)
_agg_no_counts = _make_agg(False)


def _proj_kernel(x_ref, w_ref, b_ref, o_ref):
    o_ref[...] = jax.nn.relu(
        jnp.dot(x_ref[...], w_ref[...], preferred_element_type=jnp.float32)
        + b_ref[...])


def _proj(x, w, b):
    return pl.pallas_call(
        _proj_kernel,
        grid=(10,),
        in_specs=[pl.BlockSpec((N // 10, H), lambda i: (i, 0)),
                  pl.BlockSpec((H, H), lambda i: (0, 0)),
                  pl.BlockSpec((1, H), lambda i: (0, 0))],
        out_specs=pl.BlockSpec((N // 10, H), lambda i: (i, 0)),
        out_shape=jax.ShapeDtypeStruct((N, H), jnp.float32),
    )(x, w, b.reshape(1, H))


def _comb_kernel(s_ref, cnt_ref, x_ref, wl_ref, bl_ref, wr_ref, o_ref):
    mean = s_ref[...] / jnp.maximum(cnt_ref[...], 1.0)
    o_ref[...] = (
        jnp.dot(mean, wl_ref[...], preferred_element_type=jnp.float32)
        + bl_ref[...]
        + jnp.dot(x_ref[...], wr_ref[...], preferred_element_type=jnp.float32))


def _comb(sums, cnt3, x, wl, bl, wr):
    blk = N // 10
    return pl.pallas_call(
        _comb_kernel,
        grid=(10,),
        in_specs=[pl.BlockSpec((blk, H), lambda i: (i, 0)),
                  pl.BlockSpec((blk, 1), lambda i: (i, 0)),
                  pl.BlockSpec((blk, H), lambda i: (i, 0)),
                  pl.BlockSpec((H, H), lambda i: (0, 0)),
                  pl.BlockSpec((1, H), lambda i: (0, 0)),
                  pl.BlockSpec((H, H), lambda i: (0, 0))],
        out_specs=pl.BlockSpec((blk, H), lambda i: (i, 0)),
        out_shape=jax.ShapeDtypeStruct((N, H), jnp.float32),
    )(sums, cnt3, x, wl, bl.reshape(1, H), wr)


def _final_kernel(x_ref, w_ref, b_ref, o_ref):
    o_ref[...] = (
        jnp.dot(x_ref[...], w_ref[...], preferred_element_type=jnp.float32)
        + b_ref[...])


def _final(x, w, b):
    out = w.shape[1]
    return pl.pallas_call(
        _final_kernel,
        grid=(10,),
        in_specs=[pl.BlockSpec((N // 10, H), lambda i: (i, 0)),
                  pl.BlockSpec((H, out), lambda i: (0, 0)),
                  pl.BlockSpec((1, out), lambda i: (0, 0))],
        out_specs=pl.BlockSpec((N // 10, out), lambda i: (i, 0)),
        out_shape=jax.ShapeDtypeStruct((N, out), jnp.float32),
    )(x, w, b.reshape(1, out))


def kernel(x_user, x_item, ei_user_item, ei_item_user,
           W_in_user, b_in_user, W_in_item, b_in_item,
           l0_ui_Wl, l0_ui_bl, l0_ui_Wr, l0_iu_Wl, l0_iu_bl, l0_iu_Wr,
           l1_ui_Wl, l1_ui_bl, l1_ui_Wr, l1_iu_Wl, l1_iu_bl, l1_iu_Wr,
           W_out, b_out):
    src_ui = ei_user_item[0].astype(jnp.int32).reshape(NS, NCHUNK, CH)
    dst_ui = ei_user_item[1].astype(jnp.int32).reshape(NS, NCHUNK, CH)
    src_iu = ei_item_user[0].astype(jnp.int32).reshape(NS, NCHUNK, CH)
    dst_iu = ei_item_user[1].astype(jnp.int32).reshape(NS, NCHUNK, CH)

    y_u = _proj(x_user, W_in_user, b_in_user)
    y_i = _proj(x_item, W_in_item, b_in_item)

    sums_i, sums_u, cnt_i, cnt_u = _agg_with_counts(
        y_u, y_i, src_ui, dst_ui, src_iu, dst_iu)
    cnt_i3 = cnt_i[:N].reshape(N, 1)
    cnt_u3 = cnt_u[:N].reshape(N, 1)

    new_i = _comb(sums_i[:N], cnt_i3, y_i, l0_ui_Wl, l0_ui_bl, l0_ui_Wr)
    new_u = _comb(sums_u[:N], cnt_u3, y_u, l0_iu_Wl, l0_iu_bl, l0_iu_Wr)
    y_u, y_i = new_u, new_i

    sums_i, sums_u = _agg_no_counts(y_u, y_i, src_ui, dst_ui, src_iu, dst_iu)

    new_i = _comb(sums_i[:N], cnt_i3, y_i, l1_ui_Wl, l1_ui_bl, l1_ui_Wr)
    new_u = _comb(sums_u[:N], cnt_u3, y_u, l1_iu_Wl, l1_iu_bl, l1_iu_Wr)
    y_u = new_u

    return _final(y_u, W_out, b_out)


# speedup vs baseline: 1.6764x; 1.6764x over previous
import functools

import jax
import jax.numpy as jnp
from jax import lax
from jax.experimental import pallas as pl
from jax.experimental.pallas import tpu as pltpu
from jax.experimental.pallas import tpu_sc as plsc

N = 10000
H = 128
NPAD = 10112
E = 320000
NC, NS = 2, 16
NW = NC * NS
EPW = E // NW
CH = 125
NCHUNK = EPW // CH
HC = NCHUNK // 2
RPS = NPAD // NS
CPAD = 10240
CW = CPAD // NS

_MESH = plsc.VectorSubcoreMesh(core_axis_name="c", subcore_axis_name="s")


def _agg_body(with_counts, x_hbm, src_hbm, dst_hbm, sums_out, *rest):
    if with_counts:
        (cnt_out, src_v, dst_v, rows_a, rows_b, sem, sem2,
         ones_v, cnt_sh, acc_sh) = rest
    else:
        (src_v, dst_v, rows_a, rows_b, sem, sem2, ones_v, acc_sh) = rest
    cid = lax.axis_index("c")
    sid = lax.axis_index("s")
    wid = cid * NS + sid

    zero16 = jnp.zeros((16,), jnp.float32)
    for r in range(CH):
        for k in range(H // 16):
            rows_a[r, pl.ds(k * 16, 16)] = zero16
    ZC = 120
    for b in range(RPS // ZC):
        pltpu.sync_copy(rows_a.at[pl.ds(0, ZC)],
                        acc_sh.at[pl.ds(sid * RPS + b * ZC, ZC)])
    tail = RPS % ZC
    if tail:
        pltpu.sync_copy(rows_a.at[pl.ds(0, tail)],
                        acc_sh.at[pl.ds(sid * RPS + (RPS // ZC) * ZC, tail)])

    if with_counts:
        one16 = jnp.ones((16,), jnp.float32)
        for k in range(8):
            ones_v[pl.ds(k * 16, 16)] = one16
        for b in range(CW // H):
            pltpu.sync_copy(rows_a.at[2 * b],
                            cnt_sh.at[pl.ds(sid * CW + b * H, H)])

    plsc.subcore_barrier()

    def _scat_start(j, rows):
        hs = [pltpu.async_copy(rows, acc_sh.at[dst_v.at[j]], sem2, add=True)]
        if with_counts:
            hs.append(pltpu.async_copy(ones_v.at[pl.ds(0, CH)],
                                       cnt_sh.at[dst_v.at[j]], sem2,
                                       add=True))
        return hs

    for half in range(2):
        pltpu.sync_copy(src_hbm.at[wid, pl.ds(half * HC, HC)], src_v)
        pltpu.sync_copy(dst_hbm.at[wid, pl.ds(half * HC, HC)], dst_v)

        pltpu.async_copy(x_hbm.at[src_v.at[0]], rows_a, sem).wait()

        def pair(p, carry):
            j0 = 2 * p
            hs = _scat_start(j0, rows_a)
            pltpu.async_copy(x_hbm.at[src_v.at[j0 + 1]], rows_b, sem).wait()
            for h in hs:
                h.wait()
            hs = _scat_start(j0 + 1, rows_b)
            jn = jnp.minimum(j0 + 2, HC - 1)
            pltpu.async_copy(x_hbm.at[src_v.at[jn]], rows_a, sem).wait()
            for h in hs:
                h.wait()
            return carry
        lax.fori_loop(0, HC // 2, pair, 0)

    plsc.subcore_barrier()

    pltpu.sync_copy(acc_sh.at[pl.ds(sid * RPS, RPS)],
                    sums_out.at[cid, pl.ds(sid * RPS, RPS)])
    if with_counts:
        pltpu.sync_copy(cnt_sh.at[pl.ds(sid * CW, CW)],
                        cnt_out.at[cid, pl.ds(sid * CW, CW)])


def _make_agg(with_counts):
    out_type = [jax.ShapeDtypeStruct((NC, NPAD, H), jnp.float32)]
    scratch = [
        pltpu.VMEM((HC, CH), jnp.int32),
        pltpu.VMEM((HC, CH), jnp.int32),
        pltpu.VMEM((CH, H), jnp.float32),
        pltpu.VMEM((CH, H), jnp.float32),
        pltpu.SemaphoreType.DMA,
        pltpu.SemaphoreType.DMA,
    ]
    if with_counts:
        out_type.append(jax.ShapeDtypeStruct((NC, CPAD), jnp.float32))
        scratch += [
            pltpu.VMEM((128,), jnp.float32),
            pltpu.VMEM_SHARED((CPAD,), jnp.float32),
        ]
    else:
        scratch.append(pltpu.VMEM((128,), jnp.float32))
    scratch.append(pltpu.VMEM_SHARED((NPAD, H), jnp.float32))
    return pl.kernel(
        functools.partial(_agg_body, with_counts),
        out_type=tuple(out_type),
        mesh=_MESH,
        scratch_types=tuple(scratch),
    )


_agg_with_counts = _make_agg(True)
_agg_no_counts = _make_agg(False)


def _proj_kernel(x_ref, w_ref, b_ref, o_ref):
    o_ref[...] = jax.nn.relu(
        jnp.dot(x_ref[...], w_ref[...], preferred_element_type=jnp.float32)
        + b_ref[...])


def _proj(x, w, b):
    return pl.pallas_call(
        _proj_kernel,
        grid=(10,),
        in_specs=[pl.BlockSpec((N // 10, H), lambda i: (i, 0)),
                  pl.BlockSpec((H, H), lambda i: (0, 0)),
                  pl.BlockSpec((1, H), lambda i: (0, 0))],
        out_specs=pl.BlockSpec((N // 10, H), lambda i: (i, 0)),
        out_shape=jax.ShapeDtypeStruct((N, H), jnp.float32),
    )(x, w, b.reshape(1, H))


def _comb_kernel(parts_ref, cnt_ref, x_ref, wl_ref, bl_ref, wr_ref, o_ref):
    s = parts_ref[0] + parts_ref[1]
    c = cnt_ref[0] + cnt_ref[1]
    mean = s / jnp.maximum(c, 1.0)
    o_ref[...] = (
        jnp.dot(mean, wl_ref[...], preferred_element_type=jnp.float32)
        + bl_ref[...]
        + jnp.dot(x_ref[...], wr_ref[...], preferred_element_type=jnp.float32))


def _comb(parts, cnt3, x, wl, bl, wr):
    blk = N // 10
    return pl.pallas_call(
        _comb_kernel,
        grid=(10,),
        in_specs=[pl.BlockSpec((NC, blk, H), lambda i: (0, i, 0)),
                  pl.BlockSpec((NC, blk, 1), lambda i: (0, i, 0)),
                  pl.BlockSpec((blk, H), lambda i: (i, 0)),
                  pl.BlockSpec((H, H), lambda i: (0, 0)),
                  pl.BlockSpec((1, H), lambda i: (0, 0)),
                  pl.BlockSpec((H, H), lambda i: (0, 0))],
        out_specs=pl.BlockSpec((blk, H), lambda i: (i, 0)),
        out_shape=jax.ShapeDtypeStruct((N, H), jnp.float32),
    )(parts, cnt3, x, wl, bl.reshape(1, H), wr)


def _final_kernel(x_ref, w_ref, b_ref, o_ref):
    o_ref[...] = (
        jnp.dot(x_ref[...], w_ref[...], preferred_element_type=jnp.float32)
        + b_ref[...])


def _final(x, w, b):
    out = w.shape[1]
    return pl.pallas_call(
        _final_kernel,
        grid=(10,),
        in_specs=[pl.BlockSpec((N // 10, H), lambda i: (i, 0)),
                  pl.BlockSpec((H, out), lambda i: (0, 0)),
                  pl.BlockSpec((1, out), lambda i: (0, 0))],
        out_specs=pl.BlockSpec((N // 10, out), lambda i: (i, 0)),
        out_shape=jax.ShapeDtypeStruct((N, out), jnp.float32),
    )(x, w, b.reshape(1, out))


def kernel(x_user, x_item, ei_user_item, ei_item_user,
           W_in_user, b_in_user, W_in_item, b_in_item,
           l0_ui_Wl, l0_ui_bl, l0_ui_Wr, l0_iu_Wl, l0_iu_bl, l0_iu_Wr,
           l1_ui_Wl, l1_ui_bl, l1_ui_Wr, l1_iu_Wl, l1_iu_bl, l1_iu_Wr,
           W_out, b_out):
    src_ui = ei_user_item[0].astype(jnp.int32).reshape(NW, NCHUNK, CH)
    dst_ui = ei_user_item[1].astype(jnp.int32).reshape(NW, NCHUNK, CH)
    src_iu = ei_item_user[0].astype(jnp.int32).reshape(NW, NCHUNK, CH)
    dst_iu = ei_item_user[1].astype(jnp.int32).reshape(NW, NCHUNK, CH)

    y_u = _proj(x_user, W_in_user, b_in_user)
    y_i = _proj(x_item, W_in_item, b_in_item)

    sums_ui, cnt_ui = _agg_with_counts(y_u, src_ui, dst_ui)
    sums_iu, cnt_iu = _agg_with_counts(y_i, src_iu, dst_iu)
    cnt_ui3 = cnt_ui[:, :N].reshape(NC, N, 1)
    cnt_iu3 = cnt_iu[:, :N].reshape(NC, N, 1)

    new_i = _comb(sums_ui[:, :N], cnt_ui3, y_i, l0_ui_Wl, l0_ui_bl, l0_ui_Wr)
    new_u = _comb(sums_iu[:, :N], cnt_iu3, y_u, l0_iu_Wl, l0_iu_bl, l0_iu_Wr)
    y_u, y_i = new_u, new_i

    (sums_ui,) = _agg_no_counts(y_u, src_ui, dst_ui)
    (sums_iu,) = _agg_no_counts(y_i, src_iu, dst_iu)

    new_i = _comb(sums_ui[:, :N], cnt_ui3, y_i, l1_ui_Wl, l1_ui_bl, l1_ui_Wr)
    new_u = _comb(sums_iu[:, :N], cnt_iu3, y_u, l1_iu_Wl, l1_iu_bl, l1_iu_Wr)
    y_u = new_u

    return _final(y_u, W_out, b_out)
